# baseline (speedup 1.0000x reference)
import jax
import jax.numpy as jnp
from jax import lax
from jax.experimental import pallas as pl
from jax.experimental.pallas import tpu as pltpu
from jax.experimental.pallas import tpu_sc as plsc

_PLANE_RES = 1024
_GRID_RES = 256
_FEAT = 4
_NC = 2
_NS = 16
_NW = _NC * _NS
_L = 16
_CHUNK = 128


def _floorfrac(v, res):
    p = v * jnp.float32(res - 1) + jnp.float32(0.5)
    i = p.astype(jnp.int32)
    f = p - i.astype(jnp.float32)
    ic = jnp.minimum(jnp.maximum(i, 0), res - 2)
    return ic, f


class _Set:
    def __init__(self, s):
        (self.x, self.y, self.z, self.fr, self.hb) = s[0:5]
        self.pidx = s[5:17]
        self.gidx = s[17:25]
        (self.prow, self.grow, self.xsem, self.gsem) = s[25:29]


_SET_LEN = 29


def _set_types():
    return (
        [pltpu.VMEM((_CHUNK,), jnp.float32)] * 3
        + [pltpu.VMEM((6 * _CHUNK,), jnp.float32)]
        + [pltpu.VMEM((4 * _CHUNK,), jnp.int32)]
        + [pltpu.VMEM((_CHUNK,), jnp.int32)] * 20
        + [pltpu.VMEM((12, _CHUNK, 2 * _FEAT), jnp.float32),
           pltpu.VMEM((8, _CHUNK, 2 * _FEAT), jnp.float32),
           pltpu.SemaphoreType.DMA,
           pltpu.SemaphoreType.DMA]
    )


def _body(x_hbm, y_hbm, z_hbm, plane_hbm, grid_hbm, out_hbm, *s):
    S0 = _Set(s[0:_SET_LEN])
    S1 = _Set(s[_SET_LEN:2 * _SET_LEN])
    out0, osem0, out1, osem1 = s[2 * _SET_LEN:2 * _SET_LEN + 4]

    wid = lax.axis_index("s") * _NC + lax.axis_index("c")
    npts = x_hbm.shape[0] // _NW
    nchunk = npts // _CHUNK
    last = nchunk - 1
    base = wid * npts

    lane = lax.iota(jnp.int32, _L)
    r4b = lane >> 2
    f4 = lane & 3
    sbase = r4b * _L + f4

    def issue_xyz(S, ci):
        off = base + ci * _CHUNK
        pltpu.async_copy(x_hbm.at[pl.ds(off, _CHUNK)], S.x, S.xsem)
        pltpu.async_copy(y_hbm.at[pl.ds(off, _CHUNK)], S.y, S.xsem)
        pltpu.async_copy(z_hbm.at[pl.ds(off, _CHUNK)], S.z, S.xsem)

    def wait_xyz(S):
        for r in (S.x, S.y, S.z):
            pltpu.make_async_copy(x_hbm.at[pl.ds(0, _CHUNK)], r, S.xsem).wait()

    def fire(S):
        for g in range(_CHUNK // _L):
            sl = pl.ds(g * _L, _L)
            x = S.x[sl]
            y = S.y[sl]
            z = S.z[sl]
            px0, pfx = _floorfrac(x, _PLANE_RES)
            py0, pfy = _floorfrac(y, _PLANE_RES)
            pz0, pfz = _floorfrac(z, _PLANE_RES)
            gx0, gfx = _floorfrac(x, _GRID_RES)
            gy0, gfy = _floorfrac(y, _GRID_RES)
            gz0, gfz = _floorfrac(z, _GRID_RES)
            S.fr[pl.ds(0 * _CHUNK + g * _L, _L)] = pfx
            S.fr[pl.ds(1 * _CHUNK + g * _L, _L)] = pfy
            S.fr[pl.ds(2 * _CHUNK + g * _L, _L)] = pfz
            S.fr[pl.ds(3 * _CHUNK + g * _L, _L)] = gfx
            S.fr[pl.ds(4 * _CHUNK + g * _L, _L)] = gfy
            S.fr[pl.ds(5 * _CHUNK + g * _L, _L)] = gfz

            R = _PLANE_RES
            b0 = px0 + py0 * R
            b1 = py0 + pz0 * R + R * R
            b2 = pz0 + px0 * R + 2 * R * R
            for pi, b in enumerate((b0, b1, b2)):
                S.pidx[4 * pi + 0][sl] = b >> 1
                S.pidx[4 * pi + 1][sl] = (b + 1) >> 1
                S.pidx[4 * pi + 2][sl] = (b + R) >> 1
                S.pidx[4 * pi + 3][sl] = (b + R + 1) >> 1
                S.hb[pl.ds(pi * _CHUNK + g * _L, _L)] = (b & 1) << 2

            G = _GRID_RES
            gb = gx0 + gy0 * G + gz0 * G * G
            for c in range(8):
                dx, dy, dz = c & 1, (c >> 1) & 1, (c >> 2) & 1
                S.gidx[c][sl] = (gb + (dx + dy * G + dz * G * G)) >> 1
            S.hb[pl.ds(3 * _CHUNK + g * _L, _L)] = (gb & 1) << 2

        for c in range(12):
            for hh in range(4):
                pltpu.async_copy(
                    plane_hbm.at[S.pidx[c].at[pl.ds(hh * 32, 32)]],
                    S.prow.at[c].at[pl.ds(hh * 32, 32)], S.gsem)
        for c in range(8):
            for hh in range(4):
                pltpu.async_copy(
                    grid_hbm.at[S.gidx[c].at[pl.ds(hh * 32, 32)]],
                    S.grow.at[c].at[pl.ds(hh * 32, 32)], S.gsem)

    def drain_gathers(S):
        for c in range(12):
            for hh in range(4):
                pltpu.make_async_copy(
                    plane_hbm.at[S.pidx[c].at[pl.ds(hh * 32, 32)]],
                    S.prow.at[c].at[pl.ds(hh * 32, 32)], S.gsem).wait()
        for c in range(8):
            for hh in range(4):
                pltpu.make_async_copy(
                    grid_hbm.at[S.gidx[c].at[pl.ds(hh * 32, 32)]],
                    S.grow.at[c].at[pl.ds(hh * 32, 32)], S.gsem).wait()

    def compute(S, out_v, osem, ci):
        @plsc.parallel_loop(0, _CHUNK // 4)
        def accum4(j):
            r4 = r4b + 4 * j

            def frac(row):
                return plsc.load_gather(S.fr, [row * _CHUNK + r4])

            pfx, pfy, pfz = frac(0), frac(1), frac(2)
            gfx, gfy, gfz = frac(3), frac(4), frac(5)
            one = jnp.float32(1.0)
            four = jnp.int32(4)
            opx, opy, opz = one - pfx, one - pfy, one - pfz
            ogx, ogy, ogz = one - gfx, one - gfy, one - gfz

            hs = [plsc.load_gather(S.hb, [k * _CHUNK + r4]) for k in range(4)]
            fA = [h + f4 for h in hs]
            fB = [(four - h) + f4 for h in hs]

            def row(ref, c, fidx):
                cc = jnp.full((_L,), c, jnp.int32)
                return plsc.load_gather(ref, [cc, r4, fidx])

            pw = (
                opx * opy, pfx * opy, opx * pfy, pfx * pfy,
                opy * opz, pfy * opz, opy * pfz, pfy * pfz,
                opz * opx, pfz * opx, opz * pfx, pfz * pfx,
            )
            for blk in range(3):
                acc = pw[4 * blk] * row(S.prow, 4 * blk, fA[blk])
                acc = acc + pw[4 * blk + 1] * row(S.prow, 4 * blk + 1, fB[blk])
                acc = acc + pw[4 * blk + 2] * row(S.prow, 4 * blk + 2, fA[blk])
                acc = acc + pw[4 * blk + 3] * row(S.prow, 4 * blk + 3, fB[blk])
                plsc.store_scatter(out_v, [sbase + (64 * j + 4 * blk)], acc)

            wxy = (ogx * ogy, gfx * ogy, ogx * gfy, gfx * gfy)
            gacc = (wxy[0] * ogz) * row(S.grow, 0, fA[3])
            for c in range(1, 8):
                w = wxy[c & 3] * (gfz if c >= 4 else ogz)
                gacc = gacc + w * row(S.grow, c, fB[3] if (c & 1) else fA[3])
            plsc.store_scatter(out_v, [sbase + (64 * j + 12)], gacc)

        off = base + ci * _CHUNK
        pltpu.async_copy(out_v, out_hbm.at[pl.ds(off * _L, _CHUNK * _L)], osem)

    def wait_out(out_v, osem):
        pltpu.make_async_copy(
            out_v, out_hbm.at[pl.ds(0, _CHUNK * _L)], osem).wait()

    issue_xyz(S0, 0)
    wait_xyz(S0)
    fire(S0)
    issue_xyz(S1, 1)

    def step(i, carry):
        c0 = 2 * i
        c1 = c0 + 1
        wait_xyz(S1)
        fire(S1)
        issue_xyz(S0, jnp.minimum(c0 + 2, last))
        drain_gathers(S0)

        @pl.when(i > 0)
        def _():
            wait_out(out0, osem0)

        compute(S0, out0, osem0, c0)
        wait_xyz(S0)
        fire(S0)
        issue_xyz(S1, jnp.minimum(c1 + 2, last))
        drain_gathers(S1)

        @pl.when(i > 0)
        def _():
            wait_out(out1, osem1)

        compute(S1, out1, osem1, c1)
        return carry

    lax.fori_loop(0, nchunk // 2, step, 0)

    drain_gathers(S0)
    wait_xyz(S1)
    wait_out(out0, osem0)
    wait_out(out1, osem1)


def kernel(xyzs, plane_embedding, grid_embedding):
    B = xyzs.shape[0]
    xt = xyzs.T
    x, y, z = xt[0], xt[1], xt[2]
    planes = plane_embedding.reshape(3 * _PLANE_RES * _PLANE_RES // 2, 2 * _FEAT)
    grid = grid_embedding.reshape(_GRID_RES ** 3 // 2, 2 * _FEAT)

    mesh = plsc.VectorSubcoreMesh(core_axis_name="c", subcore_axis_name="s")
    run = pl.kernel(
        _body,
        out_type=jax.ShapeDtypeStruct((B * 16,), jnp.float32),
        mesh=mesh,
        compiler_params=pltpu.CompilerParams(
            needs_layout_passes=False, use_tc_tiling_on_sc=False),
        scratch_types=(
            _set_types() + _set_types()
            + [pltpu.VMEM((_CHUNK * _L,), jnp.float32),
               pltpu.SemaphoreType.DMA,
               pltpu.VMEM((_CHUNK * _L,), jnp.float32),
               pltpu.SemaphoreType.DMA]
        ),
    )
    out = run(x, y, z, planes, grid)
    return out.reshape(B, 16)


# speedup vs baseline: 1.1642x; 1.1642x over previous
import jax
import jax.numpy as jnp
from jax import lax
from jax.experimental import pallas as pl
from jax.experimental.pallas import tpu as pltpu
from jax.experimental.pallas import tpu_sc as plsc

_PLANE_RES = 1024
_GRID_RES = 256
_FEAT = 4
_NC = 2
_NS = 16
_NW = _NC * _NS
_L = 16
_CHUNK = 128


def _floorfrac(v, res):
    p = v * jnp.float32(res - 1) + jnp.float32(0.5)
    i = p.astype(jnp.int32)
    f = p - i.astype(jnp.float32)
    ic = jnp.minimum(jnp.maximum(i, 0), res - 2)
    return ic, f


class _Set:
    def __init__(self, s):
        (self.x, self.y, self.z, self.fr, self.hb) = s[0:5]
        self.pidx = s[5:17]
        self.gidx = s[17:25]
        (self.prow, self.grow, self.xsem, self.gsem) = s[25:29]


_SET_LEN = 29


def _set_types():
    return (
        [pltpu.VMEM((_CHUNK,), jnp.float32)] * 3
        + [pltpu.VMEM((6 * _CHUNK,), jnp.float32)]
        + [pltpu.VMEM((4 * _CHUNK,), jnp.int32)]
        + [pltpu.VMEM((_CHUNK,), jnp.int32)] * 20
        + [pltpu.VMEM((12, _CHUNK, 2 * _FEAT), jnp.float32),
           pltpu.VMEM((8, _CHUNK, 2 * _FEAT), jnp.float32),
           pltpu.SemaphoreType.DMA,
           pltpu.SemaphoreType.DMA]
    )


def _body(x_hbm, y_hbm, z_hbm, plane_hbm, grid_hbm, out_hbm, *s):
    S0 = _Set(s[0:_SET_LEN])
    S1 = _Set(s[_SET_LEN:2 * _SET_LEN])
    out0, osem0, out1, osem1 = s[2 * _SET_LEN:2 * _SET_LEN + 4]

    wid = lax.axis_index("s") * _NC + lax.axis_index("c")
    npts = x_hbm.shape[0] // _NW
    nchunk = npts // _CHUNK
    last = nchunk - 1
    base = wid * npts

    lane = lax.iota(jnp.int32, _L)
    r4b = lane >> 2
    f4 = lane & 3
    sbase = r4b * _L + f4

    def issue_xyz(S, ci):
        off = base + ci * _CHUNK
        pltpu.async_copy(x_hbm.at[pl.ds(off, _CHUNK)], S.x, S.xsem)
        pltpu.async_copy(y_hbm.at[pl.ds(off, _CHUNK)], S.y, S.xsem)
        pltpu.async_copy(z_hbm.at[pl.ds(off, _CHUNK)], S.z, S.xsem)

    def wait_xyz(S):
        for r in (S.x, S.y, S.z):
            pltpu.make_async_copy(x_hbm.at[pl.ds(0, _CHUNK)], r, S.xsem).wait()

    def fire(S):
        for g in range(_CHUNK // _L):
            sl = pl.ds(g * _L, _L)
            x = S.x[sl]
            y = S.y[sl]
            z = S.z[sl]
            px0, pfx = _floorfrac(x, _PLANE_RES)
            py0, pfy = _floorfrac(y, _PLANE_RES)
            pz0, pfz = _floorfrac(z, _PLANE_RES)
            gx0, gfx = _floorfrac(x, _GRID_RES)
            gy0, gfy = _floorfrac(y, _GRID_RES)
            gz0, gfz = _floorfrac(z, _GRID_RES)
            S.fr[pl.ds(0 * _CHUNK + g * _L, _L)] = pfx
            S.fr[pl.ds(1 * _CHUNK + g * _L, _L)] = pfy
            S.fr[pl.ds(2 * _CHUNK + g * _L, _L)] = pfz
            S.fr[pl.ds(3 * _CHUNK + g * _L, _L)] = gfx
            S.fr[pl.ds(4 * _CHUNK + g * _L, _L)] = gfy
            S.fr[pl.ds(5 * _CHUNK + g * _L, _L)] = gfz

            R = _PLANE_RES
            b0 = px0 + py0 * R
            b1 = py0 + pz0 * R + R * R
            b2 = pz0 + px0 * R + 2 * R * R
            for pi, b in enumerate((b0, b1, b2)):
                S.pidx[4 * pi + 0][sl] = b >> 1
                S.pidx[4 * pi + 1][sl] = (b + 1) >> 1
                S.pidx[4 * pi + 2][sl] = (b + R) >> 1
                S.pidx[4 * pi + 3][sl] = (b + R + 1) >> 1
                S.hb[pl.ds(pi * _CHUNK + g * _L, _L)] = (b & 1) << 2

            G = _GRID_RES
            gb = gx0 + gy0 * G + gz0 * G * G
            for c in range(8):
                dx, dy, dz = c & 1, (c >> 1) & 1, (c >> 2) & 1
                S.gidx[c][sl] = (gb + (dx + dy * G + dz * G * G)) >> 1
            S.hb[pl.ds(3 * _CHUNK + g * _L, _L)] = (gb & 1) << 2

        for c in range(12):
            for hh in range(2):
                pltpu.async_copy(
                    plane_hbm.at[S.pidx[c].at[pl.ds(hh * 64, 64)]],
                    S.prow.at[c].at[pl.ds(hh * 64, 64)], S.gsem)
        for c in range(8):
            for hh in range(2):
                pltpu.async_copy(
                    grid_hbm.at[S.gidx[c].at[pl.ds(hh * 64, 64)]],
                    S.grow.at[c].at[pl.ds(hh * 64, 64)], S.gsem)

    def drain_gathers(S):
        for c in range(12):
            for hh in range(2):
                pltpu.make_async_copy(
                    plane_hbm.at[S.pidx[c].at[pl.ds(hh * 64, 64)]],
                    S.prow.at[c].at[pl.ds(hh * 64, 64)], S.gsem).wait()
        for c in range(8):
            for hh in range(2):
                pltpu.make_async_copy(
                    grid_hbm.at[S.gidx[c].at[pl.ds(hh * 64, 64)]],
                    S.grow.at[c].at[pl.ds(hh * 64, 64)], S.gsem).wait()

    def compute(S, out_v, osem, ci):
        @plsc.parallel_loop(0, _CHUNK // 4)
        def accum4(j):
            r4 = r4b + 4 * j

            def frac(row):
                return plsc.load_gather(S.fr, [row * _CHUNK + r4])

            pfx, pfy, pfz = frac(0), frac(1), frac(2)
            gfx, gfy, gfz = frac(3), frac(4), frac(5)
            one = jnp.float32(1.0)
            four = jnp.int32(4)
            opx, opy, opz = one - pfx, one - pfy, one - pfz
            ogx, ogy, ogz = one - gfx, one - gfy, one - gfz

            hs = [plsc.load_gather(S.hb, [k * _CHUNK + r4]) for k in range(4)]
            fA = [h + f4 for h in hs]
            fB = [(four - h) + f4 for h in hs]

            def row(ref, c, fidx):
                cc = jnp.full((_L,), c, jnp.int32)
                return plsc.load_gather(ref, [cc, r4, fidx])

            pw = (
                opx * opy, pfx * opy, opx * pfy, pfx * pfy,
                opy * opz, pfy * opz, opy * pfz, pfy * pfz,
                opz * opx, pfz * opx, opz * pfx, pfz * pfx,
            )
            for blk in range(3):
                acc = pw[4 * blk] * row(S.prow, 4 * blk, fA[blk])
                acc = acc + pw[4 * blk + 1] * row(S.prow, 4 * blk + 1, fB[blk])
                acc = acc + pw[4 * blk + 2] * row(S.prow, 4 * blk + 2, fA[blk])
                acc = acc + pw[4 * blk + 3] * row(S.prow, 4 * blk + 3, fB[blk])
                plsc.store_scatter(out_v, [sbase + (64 * j + 4 * blk)], acc)

            wxy = (ogx * ogy, gfx * ogy, ogx * gfy, gfx * gfy)
            gacc = (wxy[0] * ogz) * row(S.grow, 0, fA[3])
            for c in range(1, 8):
                w = wxy[c & 3] * (gfz if c >= 4 else ogz)
                gacc = gacc + w * row(S.grow, c, fB[3] if (c & 1) else fA[3])
            plsc.store_scatter(out_v, [sbase + (64 * j + 12)], gacc)

        off = base + ci * _CHUNK
        pltpu.async_copy(out_v, out_hbm.at[pl.ds(off * _L, _CHUNK * _L)], osem)

    def wait_out(out_v, osem):
        pltpu.make_async_copy(
            out_v, out_hbm.at[pl.ds(0, _CHUNK * _L)], osem).wait()

    issue_xyz(S0, 0)
    wait_xyz(S0)
    fire(S0)
    issue_xyz(S1, 1)

    def step(i, carry):
        c0 = 2 * i
        c1 = c0 + 1
        wait_xyz(S1)
        fire(S1)
        issue_xyz(S0, jnp.minimum(c0 + 2, last))
        drain_gathers(S0)

        @pl.when(i > 0)
        def _():
            wait_out(out0, osem0)

        compute(S0, out0, osem0, c0)
        wait_xyz(S0)
        fire(S0)
        issue_xyz(S1, jnp.minimum(c1 + 2, last))
        drain_gathers(S1)

        @pl.when(i > 0)
        def _():
            wait_out(out1, osem1)

        compute(S1, out1, osem1, c1)
        return carry

    lax.fori_loop(0, nchunk // 2, step, 0)

    drain_gathers(S0)
    wait_xyz(S1)
    wait_out(out0, osem0)
    wait_out(out1, osem1)


def kernel(xyzs, plane_embedding, grid_embedding):
    B = xyzs.shape[0]
    xt = xyzs.T
    x, y, z = xt[0], xt[1], xt[2]
    planes = plane_embedding.reshape(3 * _PLANE_RES * _PLANE_RES // 2, 2 * _FEAT)
    grid = grid_embedding.reshape(_GRID_RES ** 3 // 2, 2 * _FEAT)

    mesh = plsc.VectorSubcoreMesh(core_axis_name="c", subcore_axis_name="s")
    run = pl.kernel(
        _body,
        out_type=jax.ShapeDtypeStruct((B * 16,), jnp.float32),
        mesh=mesh,
        compiler_params=pltpu.CompilerParams(
            needs_layout_passes=False, use_tc_tiling_on_sc=False),
        scratch_types=(
            _set_types() + _set_types()
            + [pltpu.VMEM((_CHUNK * _L,), jnp.float32),
               pltpu.SemaphoreType.DMA,
               pltpu.VMEM((_CHUNK * _L,), jnp.float32),
               pltpu.SemaphoreType.DMA]
        ),
    )
    out = run(x, y, z, planes, grid)
    return out.reshape(B, 16)
